# Initial kernel scaffold; baseline (speedup 1.0000x reference)
import jax
import jax.numpy as jnp
from jax.experimental import pallas as pl


def kernel(query_pts, query_times, buffer_voxel_index, static_features, dynamic_features, time_embeddings, Wq1, Wk1, Wv1, Wo1, Wq2, Wk2, Wv2, Wo2):
    raise NotImplementedError("write your pallas kernel here")


# speedup vs baseline: 3.0490x; 3.0490x over previous
import functools

import jax
import jax.numpy as jnp
import numpy as np
from jax import lax
from jax.experimental import pallas as pl
from jax.experimental.pallas import tpu as pltpu
from jax.experimental.pallas import tpu_sc as plsc

RES = 0.1
TABLE = 2 ** 20
D = 120
DP = 128
H = 8
HD = D // H
MODT = 201
M = 262144
P0 = 73856093
P1 = 19349669
P2 = 83492791

_NC = 2
_NS = 16
_L = 16
_NW = _NC * _NS
_PER_W = M // _NW
_CHUNK = 128
_NCHUNK = _PER_W // _CHUNK


def _sc_gather_build():
    mesh = plsc.VectorSubcoreMesh(core_axis_name="c", subcore_axis_name="s")

    @functools.partial(
        pl.kernel,
        mesh=mesh,
        out_type=[
            jax.ShapeDtypeStruct((M, DP), jnp.float32),
            jax.ShapeDtypeStruct((M, DP), jnp.float32),
            jax.ShapeDtypeStruct((M,), jnp.float32),
        ],
        scratch_types=[
            pltpu.VMEM((_CHUNK,), jnp.float32),
            pltpu.VMEM((_CHUNK,), jnp.float32),
            pltpu.VMEM((_CHUNK,), jnp.float32),
            pltpu.VMEM((_CHUNK,), jnp.int32),
            pltpu.VMEM((_CHUNK,), jnp.int32),
            pltpu.VMEM((_CHUNK,), jnp.int32),
            pltpu.VMEM((_CHUNK,), jnp.float32),
            pltpu.VMEM((_CHUNK, DP), jnp.float32),
            pltpu.VMEM((_CHUNK, DP), jnp.float32),
            pltpu.SemaphoreType.DMA,
            pltpu.SemaphoreType.DMA,
        ],
    )
    def sc_gather(px, py, pz, buf, statf, dynf,
                  out_s, out_d, out_v,
                  xs, ys, zs, hidx, vidx, sidx, vldf, srows, drows,
                  sem_s, sem_d):
        wid = (lax.axis_index("s") * jnp.int32(_NC)
               + lax.axis_index("c")).astype(jnp.int32)
        base = wid * jnp.int32(_PER_W)

        def chunk_body(c, carry):
            off = base + c * jnp.int32(_CHUNK)
            pltpu.sync_copy(px.at[pl.ds(off, _CHUNK)], xs)
            pltpu.sync_copy(py.at[pl.ds(off, _CHUNK)], ys)
            pltpu.sync_copy(pz.at[pl.ds(off, _CHUNK)], zs)
            for i in range(_CHUNK // _L):
                s = pl.ds(i * _L, _L)
                gx = (xs[s] / RES).astype(jnp.int32)
                gy = (ys[s] / RES).astype(jnp.int32)
                gz = (zs[s] / RES).astype(jnp.int32)
                h = (gx * P0 + gy * P1 + gz * P2) & (TABLE - 1)
                hidx[s] = h
            pltpu.async_copy(buf.at[hidx], vidx, sem_s).wait()
            for i in range(_CHUNK // _L):
                s = pl.ds(i * _L, _L)
                v = vidx[s]
                sidx[s] = jnp.maximum(v, 0)
                vldf[s] = jnp.where(v >= 0, jnp.float32(1.0), jnp.float32(0.0))
            cp_s = pltpu.async_copy(statf.at[sidx], srows, sem_s)
            cp_d = pltpu.async_copy(dynf.at[sidx], drows, sem_d)
            cp_s.wait()
            cp_d.wait()
            pltpu.sync_copy(srows, out_s.at[pl.ds(off, _CHUNK)])
            pltpu.sync_copy(drows, out_d.at[pl.ds(off, _CHUNK)])
            pltpu.sync_copy(vldf, out_v.at[pl.ds(off, _CHUNK)])
            return carry

        lax.fori_loop(jnp.int32(0), jnp.int32(_NCHUNK), chunk_body,
                      jnp.int32(0))

    return sc_gather


_sc_gather = _sc_gather_build()

_BM = 1024
_INV_SQRT_HD = np.float32(1.0) / np.sqrt(np.float32(HD))


def _tc_fusion_body(vld_ref, t_ref, stat_ref, dyn_ref, te_ref,
                    wq1, wk1, wv1, wo1, wq2, wk2, wv2, wo2, out_ref):
    f32 = jnp.float32
    sd = lax.broadcasted_iota(jnp.int32, (DP, H), 0) // HD
    sh = lax.broadcasted_iota(jnp.int32, (DP, H), 1)
    S = (sd == sh).astype(f32)
    td = lax.broadcasted_iota(jnp.int32, (H, DP), 1) // HD
    th = lax.broadcasted_iota(jnp.int32, (H, DP), 0)
    St = (td == th).astype(f32)

    def dot(a, b):
        return jnp.dot(a, b, preferred_element_type=f32)

    def fusion(a, b, Wq, Wk, Wv, Wo):
        q = dot(a, Wq[...])
        ka = dot(a, Wk[...])
        kb = dot(b, Wk[...])
        va = dot(a, Wv[...])
        vb = dot(b, Wv[...])
        l0 = dot(q * ka, S) * _INV_SQRT_HD
        l1 = dot(q * kb, S) * _INV_SQRT_HD
        w1 = 1.0 / (1.0 + jnp.exp(l0 - l1))
        w0 = 1.0 - w1
        out = dot(w0, St) * va + dot(w1, St) * vb
        return a + dot(out, Wo[...])

    tm = t_ref[...] % MODT
    oh = (tm == lax.broadcasted_iota(jnp.int32, (_BM, MODT), 1)).astype(f32)
    te = dot(oh, te_ref[...])

    dyn = dyn_ref[...]
    stat = stat_ref[...]
    cond = fusion(dyn, te, wq1, wk1, wv1, wo1)
    fused = fusion(stat, cond, wq2, wk2, wv2, wo2)
    out_ref[...] = (fused * vld_ref[...])[:, :D]


def _tc_fusion(vld, times, stat_g, dyn_g, te, weights):
    grid = (M // _BM,)
    z = np.int32(0)
    row_spec = pl.BlockSpec((_BM, DP), lambda i: (i, z))
    col1_spec = pl.BlockSpec((_BM, 1), lambda i: (i, z))
    w_spec = pl.BlockSpec((DP, DP), lambda i: (z, z))
    te_spec = pl.BlockSpec((MODT, DP), lambda i: (z, z))
    out_spec = pl.BlockSpec((_BM, D), lambda i: (i, z))
    return pl.pallas_call(
        _tc_fusion_body,
        grid=grid,
        in_specs=[col1_spec, col1_spec, row_spec, row_spec, te_spec] +
                 [w_spec] * 8,
        out_specs=out_spec,
        out_shape=jax.ShapeDtypeStruct((M, D), jnp.float32),
    )(vld, times, stat_g, dyn_g, te, *weights)


def kernel(query_pts, query_times, buffer_voxel_index, static_features,
           dynamic_features, time_embeddings,
           Wq1, Wk1, Wv1, Wo1, Wq2, Wk2, Wv2, Wo2):
    px = query_pts[:, 0]
    py = query_pts[:, 1]
    pz = query_pts[:, 2]
    buf32 = buffer_voxel_index.astype(jnp.int32)
    times32 = query_times.astype(jnp.int32).reshape(M, 1)
    pad_w = ((0, 0), (0, DP - D))
    stat_p = jnp.pad(static_features, pad_w)
    dyn_p = jnp.pad(dynamic_features, pad_w)
    te_p = jnp.pad(time_embeddings, pad_w)
    weights = [jnp.pad(w, (pad_w[1], pad_w[1]))
               for w in (Wq1, Wk1, Wv1, Wo1, Wq2, Wk2, Wv2, Wo2)]
    stat_g, dyn_g, vld = _sc_gather(px, py, pz, buf32, stat_p, dyn_p)
    out = _tc_fusion(vld.reshape(M, 1), times32, stat_g, dyn_g, te_p, weights)
    return out


# speedup vs baseline: 3.0816x; 1.0107x over previous
import functools

import jax
import jax.numpy as jnp
import numpy as np
from jax import lax
from jax.experimental import pallas as pl
from jax.experimental.pallas import tpu as pltpu
from jax.experimental.pallas import tpu_sc as plsc

RES = 0.1
TABLE = 2 ** 20
D = 120
DP = 128
H = 8
HD = D // H
MODT = 201
M = 262144
P0 = 73856093
P1 = 19349669
P2 = 83492791

_NC = 2
_NS = 16
_L = 16
_NW = _NC * _NS
_PER_W = M // _NW
_CHUNK = 128
_NCHUNK = _PER_W // _CHUNK


def _sc_gather_build(n_rows):
    mesh = plsc.VectorSubcoreMesh(core_axis_name="c", subcore_axis_name="s")
    zero_row = jnp.int32(n_rows - 1)

    @functools.partial(
        pl.kernel,
        mesh=mesh,
        out_type=[
            jax.ShapeDtypeStruct((M, DP), jnp.float32),
            jax.ShapeDtypeStruct((M, DP), jnp.float32),
        ],
        scratch_types=[
            pltpu.VMEM((_CHUNK,), jnp.float32),
            pltpu.VMEM((_CHUNK,), jnp.float32),
            pltpu.VMEM((_CHUNK,), jnp.float32),
            pltpu.VMEM((_CHUNK,), jnp.int32),
            pltpu.VMEM((_CHUNK,), jnp.int32),
            pltpu.VMEM((_CHUNK,), jnp.int32),
            pltpu.VMEM((_CHUNK, DP), jnp.float32),
            pltpu.VMEM((_CHUNK, DP), jnp.float32),
            pltpu.SemaphoreType.DMA,
            pltpu.SemaphoreType.DMA,
        ],
    )
    def sc_gather(px, py, pz, buf, statf, dynf,
                  out_s, out_d,
                  xs, ys, zs, hidx, vidx, sidx, srows, drows,
                  sem_s, sem_d):
        wid = (lax.axis_index("s") * jnp.int32(_NC)
               + lax.axis_index("c")).astype(jnp.int32)
        base = wid * jnp.int32(_PER_W)

        def chunk_body(c, carry):
            off = base + c * jnp.int32(_CHUNK)
            pltpu.sync_copy(px.at[pl.ds(off, _CHUNK)], xs)
            pltpu.sync_copy(py.at[pl.ds(off, _CHUNK)], ys)
            pltpu.sync_copy(pz.at[pl.ds(off, _CHUNK)], zs)
            for i in range(_CHUNK // _L):
                s = pl.ds(i * _L, _L)
                gx = (xs[s] / RES).astype(jnp.int32)
                gy = (ys[s] / RES).astype(jnp.int32)
                gz = (zs[s] / RES).astype(jnp.int32)
                h = (gx * P0 + gy * P1 + gz * P2) & (TABLE - 1)
                hidx[s] = h
            pltpu.async_copy(buf.at[hidx], vidx, sem_s).wait()
            for i in range(_CHUNK // _L):
                s = pl.ds(i * _L, _L)
                v = vidx[s]
                sidx[s] = jnp.where(v >= 0, v, zero_row)
            cp_s = pltpu.async_copy(statf.at[sidx], srows, sem_s)
            cp_d = pltpu.async_copy(dynf.at[sidx], drows, sem_d)
            cp_s.wait()
            cp_d.wait()
            pltpu.sync_copy(srows, out_s.at[pl.ds(off, _CHUNK)])
            pltpu.sync_copy(drows, out_d.at[pl.ds(off, _CHUNK)])
            return carry

        lax.fori_loop(jnp.int32(0), jnp.int32(_NCHUNK), chunk_body,
                      jnp.int32(0))

    return sc_gather


_BM = 1024
_INV_SQRT_HD = np.float32(1.0) / np.sqrt(np.float32(HD))


def _tc_fusion_body(t_ref, stat_ref, dyn_ref, te_ref,
                    wq1, wk1, wv1, wo1, wq2, wk2, wv2, wo2, out_ref):
    f32 = jnp.float32
    bf16 = jnp.bfloat16
    sd = lax.broadcasted_iota(jnp.int32, (DP, H), 0) // HD
    sh = lax.broadcasted_iota(jnp.int32, (DP, H), 1)
    S = (sd == sh).astype(f32)
    td = lax.broadcasted_iota(jnp.int32, (H, DP), 1) // HD
    th = lax.broadcasted_iota(jnp.int32, (H, DP), 0)
    St = (td == th).astype(f32)

    def dot(a, b):
        return jnp.dot(a.astype(bf16), b.astype(bf16),
                       preferred_element_type=f32)

    def fusion(a, b, Wq, Wk, Wv, Wo):
        ab = a.astype(bf16)
        bb = b.astype(bf16)
        q = dot(ab, Wq[...])
        ka = dot(ab, Wk[...])
        kb = dot(bb, Wk[...])
        va = dot(ab, Wv[...])
        vb = dot(bb, Wv[...])
        l0 = dot(q * ka, S) * _INV_SQRT_HD
        l1 = dot(q * kb, S) * _INV_SQRT_HD
        w1 = 1.0 / (1.0 + jnp.exp(l0 - l1))
        w0 = 1.0 - w1
        out = dot(w0, St) * va + dot(w1, St) * vb
        return a + dot(out, Wo[...])

    raw = stat_ref[...]
    vld = raw[:, 120:121]
    lane = lax.broadcasted_iota(jnp.int32, (_BM, DP), 1)
    stat = jnp.where(lane < D, raw, jnp.float32(0.0))
    dyn = dyn_ref[...]

    tm = t_ref[...].reshape(1, _BM) % MODT
    ohT = (jnp.broadcast_to(tm, (MODT, _BM))
           == lax.broadcasted_iota(jnp.int32, (MODT, _BM), 0)).astype(bf16)
    te = lax.dot_general(ohT, te_ref[...].astype(bf16),
                         (((0,), (0,)), ((), ())),
                         preferred_element_type=f32)

    cond = fusion(dyn, te, wq1, wk1, wv1, wo1)
    fused = fusion(stat, cond, wq2, wk2, wv2, wo2)
    out_ref[...] = (fused * vld)[:, :D]


def _tc_fusion(t2, stat_g, dyn_g, te, weights):
    grid = (M // _BM,)
    z = np.int32(0)
    row_spec = pl.BlockSpec((_BM, DP), lambda i: (i, z))
    t_spec = pl.BlockSpec((1, 1, _BM), lambda i: (i, z, z))
    w_spec = pl.BlockSpec((DP, DP), lambda i: (z, z))
    te_spec = pl.BlockSpec((MODT, DP), lambda i: (z, z))
    out_spec = pl.BlockSpec((_BM, D), lambda i: (i, z))
    return pl.pallas_call(
        _tc_fusion_body,
        grid=grid,
        in_specs=[t_spec, row_spec, row_spec, te_spec] + [w_spec] * 8,
        out_specs=out_spec,
        out_shape=jax.ShapeDtypeStruct((M, D), jnp.float32),
    )(t2, stat_g, dyn_g, te, *weights)


def kernel(query_pts, query_times, buffer_voxel_index, static_features,
           dynamic_features, time_embeddings,
           Wq1, Wk1, Wv1, Wo1, Wq2, Wk2, Wv2, Wo2):
    px = query_pts[:, 0]
    py = query_pts[:, 1]
    pz = query_pts[:, 2]
    t2 = query_times.astype(jnp.int32).reshape(M // _BM, 1, _BM)
    buf32 = buffer_voxel_index.astype(jnp.int32)
    nv = static_features.shape[0]
    stat_p = jnp.concatenate([
        jnp.pad(static_features, ((0, 0), (0, 1)), constant_values=1.0),
        jnp.zeros((nv, DP - D - 1), jnp.float32),
    ], axis=1)
    stat_p = jnp.concatenate([stat_p, jnp.zeros((1, DP), jnp.float32)],
                             axis=0)
    dyn_p = jnp.pad(dynamic_features, ((0, 1), (0, DP - D)))
    pad_w = ((0, 0), (0, DP - D))
    te_p = jnp.pad(time_embeddings, pad_w)
    weights = [jnp.pad(w, (pad_w[1], pad_w[1]))
               for w in (Wq1, Wk1, Wv1, Wo1, Wq2, Wk2, Wv2, Wo2)]
    stat_g, dyn_g = _sc_gather_build(nv + 1)(px, py, pz, buf32, stat_p, dyn_p)
    return _tc_fusion(t2, stat_g, dyn_g, te_p, weights)


# speedup vs baseline: 3.1476x; 1.0214x over previous
import functools

import jax
import jax.numpy as jnp
import numpy as np
from jax import lax
from jax.experimental import pallas as pl
from jax.experimental.pallas import tpu as pltpu
from jax.experimental.pallas import tpu_sc as plsc

RES = 0.1
TABLE = 2 ** 20
D = 120
DP = 128
H = 8
HD = D // H
MODT = 201
M = 262144
P0 = 73856093
P1 = 19349669
P2 = 83492791

_NC = 2
_NS = 16
_L = 16
_NW = _NC * _NS
_PER_W = M // _NW
_CHUNK = 128
_NCHUNK = _PER_W // _CHUNK


def _sc_gather_build():
    mesh = plsc.VectorSubcoreMesh(core_axis_name="c", subcore_axis_name="s")

    @functools.partial(
        pl.kernel,
        mesh=mesh,
        out_type=[
            jax.ShapeDtypeStruct((M, DP), jnp.float32),
            jax.ShapeDtypeStruct((M, DP), jnp.float32),
            jax.ShapeDtypeStruct((M,), jnp.int32),
        ],
        scratch_types=[
            pltpu.VMEM((_CHUNK,), jnp.float32),
            pltpu.VMEM((_CHUNK,), jnp.float32),
            pltpu.VMEM((_CHUNK,), jnp.float32),
            pltpu.VMEM((_CHUNK,), jnp.int32),
            pltpu.VMEM((_CHUNK,), jnp.int32),
            pltpu.VMEM((_CHUNK,), jnp.int32),
            pltpu.VMEM((_CHUNK,), jnp.int32),
            pltpu.VMEM((_CHUNK,), jnp.int32),
            pltpu.VMEM((_CHUNK, DP), jnp.float32),
            pltpu.VMEM((_CHUNK, DP), jnp.float32),
            pltpu.SemaphoreType.DMA,
            pltpu.SemaphoreType.DMA,
        ],
    )
    def sc_gather(px, py, pz, tq, buf, statf, dynf,
                  out_s, out_d, out_t,
                  xs, ys, zs, ts, hidx, vidx, sidx, tenc, srows, drows,
                  sem_s, sem_d):
        wid = (lax.axis_index("s") * jnp.int32(_NC)
               + lax.axis_index("c")).astype(jnp.int32)
        base = wid * jnp.int32(_PER_W)

        def chunk_body(c, carry):
            off = base + c * jnp.int32(_CHUNK)
            pltpu.sync_copy(px.at[pl.ds(off, _CHUNK)], xs)
            pltpu.sync_copy(py.at[pl.ds(off, _CHUNK)], ys)
            pltpu.sync_copy(pz.at[pl.ds(off, _CHUNK)], zs)
            pltpu.sync_copy(tq.at[pl.ds(off, _CHUNK)], ts)
            for i in range(_CHUNK // _L):
                s = pl.ds(i * _L, _L)
                gx = (xs[s] / RES).astype(jnp.int32)
                gy = (ys[s] / RES).astype(jnp.int32)
                gz = (zs[s] / RES).astype(jnp.int32)
                h = (gx * P0 + gy * P1 + gz * P2) & (TABLE - 1)
                hidx[s] = h
            pltpu.async_copy(buf.at[hidx], vidx, sem_s).wait()
            for i in range(_CHUNK // _L):
                s = pl.ds(i * _L, _L)
                v = vidx[s]
                sidx[s] = jnp.maximum(v, jnp.int32(0))
                tm = jnp.remainder(ts[s], jnp.int32(MODT))
                tenc[s] = jnp.where(v >= 0, tm, jnp.int32(MODT))
            cp_s = pltpu.async_copy(statf.at[sidx], srows, sem_s)
            cp_d = pltpu.async_copy(dynf.at[sidx], drows, sem_d)
            cp_s.wait()
            cp_d.wait()
            pltpu.sync_copy(srows, out_s.at[pl.ds(off, _CHUNK)])
            pltpu.sync_copy(drows, out_d.at[pl.ds(off, _CHUNK)])
            pltpu.sync_copy(tenc, out_t.at[pl.ds(off, _CHUNK)])
            return carry

        lax.fori_loop(jnp.int32(0), jnp.int32(_NCHUNK), chunk_body,
                      jnp.int32(0))

    return sc_gather


_sc_gather = _sc_gather_build()

_BM = 1024
_INV_SQRT_HD = np.float32(1.0) / np.sqrt(np.float32(HD))
_TE_ROWS = MODT + 1


def _tc_fusion_body(t_ref, stat_ref, dyn_ref, te_ref,
                    wq1, wk1, wv1, wo1, wq2, wk2, wv2, wo2, out_ref):
    f32 = jnp.float32
    bf16 = jnp.bfloat16
    sd = lax.broadcasted_iota(jnp.int32, (DP, H), 0) // HD
    sh = lax.broadcasted_iota(jnp.int32, (DP, H), 1)
    S = (sd == sh).astype(bf16)
    td = lax.broadcasted_iota(jnp.int32, (H, DP), 1) // HD
    th = lax.broadcasted_iota(jnp.int32, (H, DP), 0)
    St = (td == th).astype(bf16)

    def fusion(a, b, Wq, Wk, Wv, Wo):
        ab = a.astype(bf16)
        bb = b.astype(bf16)

        def dotb(x, y):
            return jnp.dot(x, y, preferred_element_type=f32).astype(bf16)

        q = dotb(ab, Wq[...])
        ka = dotb(ab, Wk[...])
        kb = dotb(bb, Wk[...])
        va = dotb(ab, Wv[...])
        vb = dotb(bb, Wv[...])
        l0 = jnp.dot(q * ka, S, preferred_element_type=f32) * _INV_SQRT_HD
        l1 = jnp.dot(q * kb, S, preferred_element_type=f32) * _INV_SQRT_HD
        w1 = 1.0 / (1.0 + jnp.exp(l0 - l1))
        w0 = (1.0 - w1).astype(bf16)
        w1 = w1.astype(bf16)
        out = dotb(w0, St) * va + dotb(w1, St) * vb
        return a + jnp.dot(out, Wo[...], preferred_element_type=f32)

    tm = t_ref[...].reshape(1, _BM)
    ohT = (jnp.broadcast_to(tm, (_TE_ROWS, _BM))
           == lax.broadcasted_iota(jnp.int32, (_TE_ROWS, _BM), 0)
           ).astype(bf16)
    raw_te = lax.dot_general(ohT, te_ref[...],
                             (((0,), (0,)), ((), ())),
                             preferred_element_type=f32)
    vld = raw_te[:, 120:121]
    lane = lax.broadcasted_iota(jnp.int32, (_BM, DP), 1)
    te = jnp.where(lane == 120, jnp.float32(0.0), raw_te)

    stat = stat_ref[...]
    dyn = dyn_ref[...]
    cond = fusion(dyn, te, wq1, wk1, wv1, wo1)
    fused = fusion(stat, cond, wq2, wk2, wv2, wo2)
    out_ref[...] = (fused * vld)[:, :D]


def _tc_fusion(t3, stat_g, dyn_g, te_ext, weights):
    grid = (M // _BM,)
    z = np.int32(0)
    row_spec = pl.BlockSpec((_BM, DP), lambda i: (i, z))
    t_spec = pl.BlockSpec((1, 1, _BM), lambda i: (i, z, z))
    w_spec = pl.BlockSpec((DP, DP), lambda i: (z, z))
    te_spec = pl.BlockSpec((_TE_ROWS, DP), lambda i: (z, z))
    out_spec = pl.BlockSpec((_BM, D), lambda i: (i, z))
    return pl.pallas_call(
        _tc_fusion_body,
        grid=grid,
        in_specs=[t_spec, row_spec, row_spec, te_spec] + [w_spec] * 8,
        out_specs=out_spec,
        out_shape=jax.ShapeDtypeStruct((M, D), jnp.float32),
    )(t3, stat_g, dyn_g, te_ext, *weights)


def kernel(query_pts, query_times, buffer_voxel_index, static_features,
           dynamic_features, time_embeddings,
           Wq1, Wk1, Wv1, Wo1, Wq2, Wk2, Wv2, Wo2):
    px = query_pts[:, 0]
    py = query_pts[:, 1]
    pz = query_pts[:, 2]
    t32 = query_times.astype(jnp.int32)
    buf32 = buffer_voxel_index.astype(jnp.int32)
    pad_w = ((0, 0), (0, DP - D))
    stat_p = jnp.pad(static_features, pad_w)
    dyn_p = jnp.pad(dynamic_features, pad_w)
    te_ext = jnp.concatenate([
        jnp.pad(time_embeddings, ((0, 0), (0, 1)), constant_values=1.0),
        jnp.zeros((MODT, DP - D - 1), jnp.float32),
    ], axis=1)
    te_ext = jnp.concatenate(
        [te_ext, jnp.zeros((1, DP), jnp.float32)], axis=0).astype(jnp.bfloat16)
    weights = [jnp.pad(w, (pad_w[1], pad_w[1])).astype(jnp.bfloat16)
               for w in (Wq1, Wk1, Wv1, Wo1, Wq2, Wk2, Wv2, Wo2)]
    stat_g, dyn_g, tenc = _sc_gather(px, py, pz, t32, buf32, stat_p, dyn_p)
    t3 = tenc.reshape(M // _BM, 1, _BM)
    return _tc_fusion(t3, stat_g, dyn_g, te_ext, weights)


# speedup vs baseline: 3.5431x; 1.1256x over previous
import functools

import jax
import jax.numpy as jnp
import numpy as np
from jax import lax
from jax.experimental import pallas as pl
from jax.experimental.pallas import tpu as pltpu
from jax.experimental.pallas import tpu_sc as plsc

RES = 0.1
TABLE = 2 ** 20
D = 120
DP = 128
H = 8
HD = D // H
MODT = 201
M = 262144
P0 = 73856093
P1 = 19349669
P2 = 83492791

_NC = 2
_NS = 16
_L = 16
_NW = _NC * _NS
_PER_W = M // _NW
_CHUNK = 128
_NCHUNK = _PER_W // _CHUNK


def _sc_gather_build():
    mesh = plsc.VectorSubcoreMesh(core_axis_name="c", subcore_axis_name="s")

    @functools.partial(
        pl.kernel,
        mesh=mesh,
        out_type=[
            jax.ShapeDtypeStruct((M, DP), jnp.float32),
            jax.ShapeDtypeStruct((M, DP), jnp.float32),
            jax.ShapeDtypeStruct((M,), jnp.int32),
        ],
        scratch_types=[
            pltpu.VMEM((_PER_W,), jnp.float32),
            pltpu.VMEM((_PER_W,), jnp.int32),
            pltpu.VMEM((_PER_W,), jnp.int32),
            pltpu.VMEM((_PER_W,), jnp.int32),
            pltpu.VMEM((_PER_W,), jnp.int32),
            pltpu.VMEM((_CHUNK, DP), jnp.float32),
            pltpu.VMEM((_CHUNK, DP), jnp.float32),
            pltpu.VMEM((_CHUNK, DP), jnp.float32),
            pltpu.VMEM((_CHUNK, DP), jnp.float32),
            pltpu.SemaphoreType.DMA,
            pltpu.SemaphoreType.DMA,
            pltpu.SemaphoreType.DMA,
            pltpu.SemaphoreType.DMA,
            pltpu.SemaphoreType.DMA,
        ],
    )
    def sc_gather(px, py, pz, tq, buf, statf, dynf,
                  out_s, out_d, out_t,
                  coords, hidx, vidx, sidx, tsv,
                  srows0, srows1, drows0, drows1,
                  sem_i, sem_s0, sem_d0, sem_s1, sem_d1):
        wid = (lax.axis_index("s") * jnp.int32(_NC)
               + lax.axis_index("c")).astype(jnp.int32)
        base = wid * jnp.int32(_PER_W)
        span = pl.ds(base, _PER_W)

        def hash_pass(src, prime, first):
            pltpu.sync_copy(src.at[span], coords)

            def body(i, carry):
                s = pl.ds(i * jnp.int32(_L), _L)
                g = (coords[s] / RES).astype(jnp.int32) * jnp.int32(prime)
                hidx[s] = g if first else hidx[s] + g
                return carry

            lax.fori_loop(jnp.int32(0), jnp.int32(_PER_W // _L), body,
                          jnp.int32(0))

        hash_pass(px, P0, True)
        hash_pass(py, P1, False)
        hash_pass(pz, P2, False)

        def mask_body(i, carry):
            s = pl.ds(i * jnp.int32(_L), _L)
            hidx[s] = hidx[s] & jnp.int32(TABLE - 1)
            return carry

        lax.fori_loop(jnp.int32(0), jnp.int32(_PER_W // _L), mask_body,
                      jnp.int32(0))

        def fire_idx(j, carry):
            s = pl.ds(j * jnp.int32(_CHUNK), _CHUNK)
            pltpu.async_copy(buf.at[hidx.at[s]], vidx.at[s], sem_i)
            return carry

        lax.fori_loop(jnp.int32(0), jnp.int32(_NCHUNK), fire_idx,
                      jnp.int32(0))
        pltpu.make_async_copy(buf.at[pl.ds(jnp.int32(0), _PER_W)], vidx,
                              sem_i).wait()

        pltpu.sync_copy(tq.at[span], tsv)

        def enc_body(i, carry):
            s = pl.ds(i * jnp.int32(_L), _L)
            v = vidx[s]
            sidx[s] = jnp.maximum(v, jnp.int32(0))
            tm = jnp.remainder(tsv[s], jnp.int32(MODT))
            hidx[s] = jnp.where(v >= 0, tm, jnp.int32(MODT))
            return carry

        lax.fori_loop(jnp.int32(0), jnp.int32(_PER_W // _L), enc_body,
                      jnp.int32(0))
        pltpu.sync_copy(hidx, out_t.at[span])

        srows = (srows0, srows1)
        drows = (drows0, drows1)
        sems = ((sem_s0, sem_d0), (sem_s1, sem_d1))

        def fire(c, slot):
            s = pl.ds(c * jnp.int32(_CHUNK), _CHUNK)
            pltpu.async_copy(statf.at[sidx.at[s]], srows[slot], sems[slot][0])
            pltpu.async_copy(dynf.at[sidx.at[s]], drows[slot], sems[slot][1])

        def drain(slot):
            pltpu.make_async_copy(statf.at[pl.ds(jnp.int32(0), _CHUNK)],
                                  srows[slot], sems[slot][0]).wait()
            pltpu.make_async_copy(dynf.at[pl.ds(jnp.int32(0), _CHUNK)],
                                  drows[slot], sems[slot][1]).wait()

        def ship(c, slot):
            off = base + c * jnp.int32(_CHUNK)
            pltpu.sync_copy(srows[slot], out_s.at[pl.ds(off, _CHUNK)])
            pltpu.sync_copy(drows[slot], out_d.at[pl.ds(off, _CHUNK)])

        fire(jnp.int32(0), 0)
        fire(jnp.int32(1), 1)

        def pipe_body(k, carry):
            c0 = k * jnp.int32(2)
            drain(0)
            ship(c0, 0)

            @pl.when(k < _NCHUNK // 2 - 1)
            def _():
                fire(c0 + jnp.int32(2), 0)

            drain(1)
            ship(c0 + jnp.int32(1), 1)

            @pl.when(k < _NCHUNK // 2 - 1)
            def _():
                fire(c0 + jnp.int32(3), 1)

            return carry

        lax.fori_loop(jnp.int32(0), jnp.int32(_NCHUNK // 2), pipe_body,
                      jnp.int32(0))

    return sc_gather


_sc_gather = _sc_gather_build()

_BM = 1024
_INV_SQRT_HD = np.float32(1.0) / np.sqrt(np.float32(HD))
_TE_ROWS = MODT + 1


def _tc_fusion_body(t_ref, stat_ref, dyn_ref, te_ref,
                    wq1, wk1, wv1, wo1, wq2, wk2, wv2, wo2, out_ref):
    f32 = jnp.float32
    bf16 = jnp.bfloat16
    sd = lax.broadcasted_iota(jnp.int32, (DP, H), 0) // HD
    sh = lax.broadcasted_iota(jnp.int32, (DP, H), 1)
    S = (sd == sh).astype(bf16)
    td = lax.broadcasted_iota(jnp.int32, (H, DP), 1) // HD
    th = lax.broadcasted_iota(jnp.int32, (H, DP), 0)
    St = (td == th).astype(bf16)

    def fusion(a, b, Wq, Wk, Wv, Wo):
        ab = a.astype(bf16)
        bb = b.astype(bf16)

        def dotb(x, y):
            return jnp.dot(x, y, preferred_element_type=f32).astype(bf16)

        q = dotb(ab, Wq[...])
        ka = dotb(ab, Wk[...])
        kb = dotb(bb, Wk[...])
        va = dotb(ab, Wv[...])
        vb = dotb(bb, Wv[...])
        l0 = jnp.dot(q * ka, S, preferred_element_type=f32) * _INV_SQRT_HD
        l1 = jnp.dot(q * kb, S, preferred_element_type=f32) * _INV_SQRT_HD
        w1 = 1.0 / (1.0 + jnp.exp(l0 - l1))
        w0 = (1.0 - w1).astype(bf16)
        w1 = w1.astype(bf16)
        out = dotb(w0, St) * va + dotb(w1, St) * vb
        return a + jnp.dot(out, Wo[...], preferred_element_type=f32)

    tm = t_ref[...].reshape(1, _BM)
    ohT = (jnp.broadcast_to(tm, (_TE_ROWS, _BM))
           == lax.broadcasted_iota(jnp.int32, (_TE_ROWS, _BM), 0)
           ).astype(bf16)
    raw_te = lax.dot_general(ohT, te_ref[...],
                             (((0,), (0,)), ((), ())),
                             preferred_element_type=f32)
    vld = raw_te[:, 120:121]
    lane = lax.broadcasted_iota(jnp.int32, (_BM, DP), 1)
    te = jnp.where(lane == 120, jnp.float32(0.0), raw_te)

    stat = stat_ref[...]
    dyn = dyn_ref[...]
    cond = fusion(dyn, te, wq1, wk1, wv1, wo1)
    fused = fusion(stat, cond, wq2, wk2, wv2, wo2)
    out_ref[...] = (fused * vld)[:, :D]


def _tc_fusion(t3, stat_g, dyn_g, te_ext, weights):
    grid = (M // _BM,)
    z = np.int32(0)
    row_spec = pl.BlockSpec((_BM, DP), lambda i: (i, z))
    t_spec = pl.BlockSpec((1, 1, _BM), lambda i: (i, z, z))
    w_spec = pl.BlockSpec((DP, DP), lambda i: (z, z))
    te_spec = pl.BlockSpec((_TE_ROWS, DP), lambda i: (z, z))
    out_spec = pl.BlockSpec((_BM, D), lambda i: (i, z))
    return pl.pallas_call(
        _tc_fusion_body,
        grid=grid,
        in_specs=[t_spec, row_spec, row_spec, te_spec] + [w_spec] * 8,
        out_specs=out_spec,
        out_shape=jax.ShapeDtypeStruct((M, D), jnp.float32),
    )(t3, stat_g, dyn_g, te_ext, *weights)


def kernel(query_pts, query_times, buffer_voxel_index, static_features,
           dynamic_features, time_embeddings,
           Wq1, Wk1, Wv1, Wo1, Wq2, Wk2, Wv2, Wo2):
    px = query_pts[:, 0]
    py = query_pts[:, 1]
    pz = query_pts[:, 2]
    t32 = query_times.astype(jnp.int32)
    buf32 = buffer_voxel_index.astype(jnp.int32)
    pad_w = ((0, 0), (0, DP - D))
    stat_p = jnp.pad(static_features, pad_w)
    dyn_p = jnp.pad(dynamic_features, pad_w)
    te_ext = jnp.concatenate([
        jnp.pad(time_embeddings, ((0, 0), (0, 1)), constant_values=1.0),
        jnp.zeros((MODT, DP - D - 1), jnp.float32),
    ], axis=1)
    te_ext = jnp.concatenate(
        [te_ext, jnp.zeros((1, DP), jnp.float32)], axis=0).astype(jnp.bfloat16)
    weights = [jnp.pad(w, (pad_w[1], pad_w[1])).astype(jnp.bfloat16)
               for w in (Wq1, Wk1, Wv1, Wo1, Wq2, Wk2, Wv2, Wo2)]
    stat_g, dyn_g, tenc = _sc_gather(px, py, pz, t32, buf32, stat_p, dyn_p)
    t3 = tenc.reshape(M // _BM, 1, _BM)
    return _tc_fusion(t3, stat_g, dyn_g, te_ext, weights)


# speedup vs baseline: 3.5879x; 1.0127x over previous
import functools

import jax
import jax.numpy as jnp
import numpy as np
from jax import lax
from jax.experimental import pallas as pl
from jax.experimental.pallas import tpu as pltpu
from jax.experimental.pallas import tpu_sc as plsc

RES = 0.1
TABLE = 2 ** 20
D = 120
DP = 128
H = 8
HD = D // H
MODT = 201
M = 262144
P0 = 73856093
P1 = 19349669
P2 = 83492791

_NC = 2
_NS = 16
_L = 16
_NW = _NC * _NS
_PER_W = M // _NW
_CHUNK = 128
_NCHUNK = _PER_W // _CHUNK


def _sc_gather_build():
    mesh = plsc.VectorSubcoreMesh(core_axis_name="c", subcore_axis_name="s")

    @functools.partial(
        pl.kernel,
        mesh=mesh,
        out_type=[
            jax.ShapeDtypeStruct((M, DP), jnp.float32),
            jax.ShapeDtypeStruct((M, DP), jnp.float32),
            jax.ShapeDtypeStruct((M,), jnp.int32),
        ],
        scratch_types=[
            pltpu.VMEM((_PER_W,), jnp.float32),
            pltpu.VMEM((_PER_W,), jnp.int32),
            pltpu.VMEM((_PER_W,), jnp.int32),
            pltpu.VMEM((_PER_W,), jnp.int32),
            pltpu.VMEM((_PER_W,), jnp.int32),
            pltpu.VMEM((_CHUNK, DP), jnp.float32),
            pltpu.VMEM((_CHUNK, DP), jnp.float32),
            pltpu.VMEM((_CHUNK, DP), jnp.float32),
            pltpu.VMEM((_CHUNK, DP), jnp.float32),
            pltpu.SemaphoreType.DMA,
            pltpu.SemaphoreType.DMA,
            pltpu.SemaphoreType.DMA,
            pltpu.SemaphoreType.DMA,
            pltpu.SemaphoreType.DMA,
        ],
    )
    def sc_gather(px, py, pz, tq, buf, statf, dynf,
                  out_s, out_d, out_t,
                  coords, hidx, vidx, sidx, tsv,
                  srows0, srows1, drows0, drows1,
                  sem_i, sem_s0, sem_d0, sem_s1, sem_d1):
        wid = (lax.axis_index("s") * jnp.int32(_NC)
               + lax.axis_index("c")).astype(jnp.int32)
        base = wid * jnp.int32(_PER_W)
        span = pl.ds(base, _PER_W)

        def hash_pass(src, prime, first):
            pltpu.sync_copy(src.at[span], coords)

            def body(i, carry):
                s = pl.ds(i * jnp.int32(_L), _L)
                g = (coords[s] / RES).astype(jnp.int32) * jnp.int32(prime)
                hidx[s] = g if first else hidx[s] + g
                return carry

            lax.fori_loop(jnp.int32(0), jnp.int32(_PER_W // _L), body,
                          jnp.int32(0))

        hash_pass(px, P0, True)
        hash_pass(py, P1, False)
        hash_pass(pz, P2, False)

        def mask_body(i, carry):
            s = pl.ds(i * jnp.int32(_L), _L)
            hidx[s] = hidx[s] & jnp.int32(TABLE - 1)
            return carry

        lax.fori_loop(jnp.int32(0), jnp.int32(_PER_W // _L), mask_body,
                      jnp.int32(0))

        def fire_idx(j, carry):
            s = pl.ds(j * jnp.int32(_CHUNK), _CHUNK)
            pltpu.async_copy(buf.at[hidx.at[s]], vidx.at[s], sem_i)
            return carry

        lax.fori_loop(jnp.int32(0), jnp.int32(_NCHUNK), fire_idx,
                      jnp.int32(0))
        pltpu.make_async_copy(buf.at[pl.ds(jnp.int32(0), _PER_W)], vidx,
                              sem_i).wait()

        pltpu.sync_copy(tq.at[span], tsv)

        def enc_body(i, carry):
            s = pl.ds(i * jnp.int32(_L), _L)
            v = vidx[s]
            sidx[s] = jnp.maximum(v, jnp.int32(0))
            tm = jnp.remainder(tsv[s], jnp.int32(MODT))
            hidx[s] = jnp.where(v >= 0, tm, jnp.int32(MODT))
            return carry

        lax.fori_loop(jnp.int32(0), jnp.int32(_PER_W // _L), enc_body,
                      jnp.int32(0))
        pltpu.sync_copy(hidx, out_t.at[span])

        srows = (srows0, srows1)
        drows = (drows0, drows1)
        sems = ((sem_s0, sem_d0), (sem_s1, sem_d1))

        def fire(c, slot):
            s = pl.ds(c * jnp.int32(_CHUNK), _CHUNK)
            pltpu.async_copy(statf.at[sidx.at[s]], srows[slot], sems[slot][0])
            pltpu.async_copy(dynf.at[sidx.at[s]], drows[slot], sems[slot][1])

        def drain(slot):
            pltpu.make_async_copy(statf.at[pl.ds(jnp.int32(0), _CHUNK)],
                                  srows[slot], sems[slot][0]).wait()
            pltpu.make_async_copy(dynf.at[pl.ds(jnp.int32(0), _CHUNK)],
                                  drows[slot], sems[slot][1]).wait()

        def ship(c, slot):
            off = base + c * jnp.int32(_CHUNK)
            pltpu.sync_copy(srows[slot], out_s.at[pl.ds(off, _CHUNK)])
            pltpu.sync_copy(drows[slot], out_d.at[pl.ds(off, _CHUNK)])

        fire(jnp.int32(0), 0)
        fire(jnp.int32(1), 1)

        def pipe_body(k, carry):
            c0 = k * jnp.int32(2)
            drain(0)
            ship(c0, 0)

            @pl.when(k < _NCHUNK // 2 - 1)
            def _():
                fire(c0 + jnp.int32(2), 0)

            drain(1)
            ship(c0 + jnp.int32(1), 1)

            @pl.when(k < _NCHUNK // 2 - 1)
            def _():
                fire(c0 + jnp.int32(3), 1)

            return carry

        lax.fori_loop(jnp.int32(0), jnp.int32(_NCHUNK // 2), pipe_body,
                      jnp.int32(0))

    return sc_gather


_sc_gather = _sc_gather_build()

_BM = 1024
_INV_SQRT_HD = np.float32(1.0) / np.sqrt(np.float32(HD))
_TE_ROWS = MODT + 1


def _tc_fusion_body(t_ref, stat_ref, dyn_ref, te_ref,
                    wqkv1, wkv1, wo1, wqkv2, wkv2, wo2, out_ref):
    f32 = jnp.float32
    bf16 = jnp.bfloat16
    sd = lax.broadcasted_iota(jnp.int32, (DP, DP), 0) // HD
    sh = lax.broadcasted_iota(jnp.int32, (DP, DP), 1) // HD
    SSt = (sd == sh).astype(bf16)

    def fusion(a, b, Wqkv, Wkv, Wo):
        ab = a.astype(bf16)
        bb = b.astype(bf16)
        qkv = jnp.dot(ab, Wqkv[...], preferred_element_type=f32)
        kvb = jnp.dot(bb, Wkv[...], preferred_element_type=f32)
        q = qkv[:, :DP].astype(bf16)
        ka = qkv[:, DP:2 * DP].astype(bf16)
        va = qkv[:, 2 * DP:].astype(bf16)
        kb = kvb[:, :DP].astype(bf16)
        vb = kvb[:, DP:].astype(bf16)
        l0e = jnp.dot(q * ka, SSt, preferred_element_type=f32)
        l1e = jnp.dot(q * kb, SSt, preferred_element_type=f32)
        w1e = 1.0 / (1.0 + jnp.exp((l0e - l1e) * _INV_SQRT_HD))
        out = (w1e * (vb - va) + va).astype(bf16)
        return a + jnp.dot(out, Wo[...], preferred_element_type=f32)

    tm = t_ref[...].reshape(1, _BM)
    ohT = (jnp.broadcast_to(tm, (_TE_ROWS, _BM))
           == lax.broadcasted_iota(jnp.int32, (_TE_ROWS, _BM), 0)
           ).astype(bf16)
    raw_te = lax.dot_general(ohT, te_ref[...],
                             (((0,), (0,)), ((), ())),
                             preferred_element_type=f32)
    vld = raw_te[:, 120:121]
    lane = lax.broadcasted_iota(jnp.int32, (_BM, DP), 1)
    te = jnp.where(lane == 120, jnp.float32(0.0), raw_te)

    stat = stat_ref[...]
    dyn = dyn_ref[...]
    cond = fusion(dyn, te, wqkv1, wkv1, wo1)
    fused = fusion(stat, cond, wqkv2, wkv2, wo2)
    out_ref[...] = (fused * vld)[:, :D]


def _tc_fusion(t3, stat_g, dyn_g, te_ext, weights):
    grid = (M // _BM,)
    z = np.int32(0)
    row_spec = pl.BlockSpec((_BM, DP), lambda i: (i, z))
    t_spec = pl.BlockSpec((1, 1, _BM), lambda i: (i, z, z))
    te_spec = pl.BlockSpec((_TE_ROWS, DP), lambda i: (z, z))
    w_specs = [pl.BlockSpec(w.shape, lambda i: (z, z)) for w in weights]
    out_spec = pl.BlockSpec((_BM, D), lambda i: (i, z))
    return pl.pallas_call(
        _tc_fusion_body,
        grid=grid,
        in_specs=[t_spec, row_spec, row_spec, te_spec] + w_specs,
        out_specs=out_spec,
        out_shape=jax.ShapeDtypeStruct((M, D), jnp.float32),
    )(t3, stat_g, dyn_g, te_ext, *weights)


def kernel(query_pts, query_times, buffer_voxel_index, static_features,
           dynamic_features, time_embeddings,
           Wq1, Wk1, Wv1, Wo1, Wq2, Wk2, Wv2, Wo2):
    px = query_pts[:, 0]
    py = query_pts[:, 1]
    pz = query_pts[:, 2]
    t32 = query_times.astype(jnp.int32)
    buf32 = buffer_voxel_index.astype(jnp.int32)
    pad_w = ((0, 0), (0, DP - D))
    stat_p = jnp.pad(static_features, pad_w)
    dyn_p = jnp.pad(dynamic_features, pad_w)
    te_ext = jnp.concatenate([
        jnp.pad(time_embeddings, ((0, 0), (0, 1)), constant_values=1.0),
        jnp.zeros((MODT, DP - D - 1), jnp.float32),
    ], axis=1)
    te_ext = jnp.concatenate(
        [te_ext, jnp.zeros((1, DP), jnp.float32)], axis=0).astype(jnp.bfloat16)
    wp = {k: jnp.pad(w, (pad_w[1], pad_w[1])).astype(jnp.bfloat16)
          for k, w in dict(q1=Wq1, k1=Wk1, v1=Wv1, o1=Wo1,
                           q2=Wq2, k2=Wk2, v2=Wv2, o2=Wo2).items()}
    weights = [
        jnp.concatenate([wp["q1"], wp["k1"], wp["v1"]], axis=1),
        jnp.concatenate([wp["k1"], wp["v1"]], axis=1),
        wp["o1"],
        jnp.concatenate([wp["q2"], wp["k2"], wp["v2"]], axis=1),
        jnp.concatenate([wp["k2"], wp["v2"]], axis=1),
        wp["o2"],
    ]
    stat_g, dyn_g, tenc = _sc_gather(px, py, pz, t32, buf32, stat_p, dyn_p)
    t3 = tenc.reshape(M // _BM, 1, _BM)
    return _tc_fusion(t3, stat_g, dyn_g, te_ext, weights)


# speedup vs baseline: 3.8939x; 1.0853x over previous
import functools

import jax
import jax.numpy as jnp
import numpy as np
from jax import lax
from jax.experimental import pallas as pl
from jax.experimental.pallas import tpu as pltpu
from jax.experimental.pallas import tpu_sc as plsc

RES = 0.1
TABLE = 2 ** 20
D = 120
DP = 128
H = 8
HD = D // H
MODT = 201
M = 262144
P0 = 73856093
P1 = 19349669
P2 = 83492791

_NC = 2
_NS = 16
_L = 16
_NW = _NC * _NS
_HALF = M // 2
_PER_W = _HALF // _NW
_CHUNK = 128
_NCHUNK = _PER_W // _CHUNK


def _sc_gather_build():
    mesh = plsc.VectorSubcoreMesh(core_axis_name="c", subcore_axis_name="s")

    @functools.partial(
        pl.kernel,
        mesh=mesh,
        out_type=[
            jax.ShapeDtypeStruct((_HALF, DP), jnp.float32),
            jax.ShapeDtypeStruct((_HALF, DP), jnp.float32),
            jax.ShapeDtypeStruct((_HALF,), jnp.int32),
        ],
        scratch_types=[
            pltpu.VMEM((_PER_W,), jnp.float32),
            pltpu.VMEM((_PER_W,), jnp.int32),
            pltpu.VMEM((_PER_W,), jnp.int32),
            pltpu.VMEM((_PER_W,), jnp.int32),
            pltpu.VMEM((_PER_W,), jnp.int32),
            pltpu.VMEM((_CHUNK, DP), jnp.float32),
            pltpu.VMEM((_CHUNK, DP), jnp.float32),
            pltpu.VMEM((_CHUNK, DP), jnp.float32),
            pltpu.VMEM((_CHUNK, DP), jnp.float32),
            pltpu.SemaphoreType.DMA,
            pltpu.SemaphoreType.DMA,
            pltpu.SemaphoreType.DMA,
            pltpu.SemaphoreType.DMA,
            pltpu.SemaphoreType.DMA,
        ],
    )
    def sc_gather(px, py, pz, tq, buf, statf, dynf,
                  out_s, out_d, out_t,
                  coords, hidx, vidx, sidx, tsv,
                  srows0, srows1, drows0, drows1,
                  sem_i, sem_s0, sem_d0, sem_s1, sem_d1):
        wid = (lax.axis_index("s") * jnp.int32(_NC)
               + lax.axis_index("c")).astype(jnp.int32)
        base = wid * jnp.int32(_PER_W)
        span = pl.ds(base, _PER_W)

        def hash_pass(src, prime, first):
            pltpu.sync_copy(src.at[span], coords)

            def body(i, carry):
                s = pl.ds(i * jnp.int32(_L), _L)
                g = (coords[s] / RES).astype(jnp.int32) * jnp.int32(prime)
                hidx[s] = g if first else hidx[s] + g
                return carry

            lax.fori_loop(jnp.int32(0), jnp.int32(_PER_W // _L), body,
                          jnp.int32(0))

        hash_pass(px, P0, True)
        hash_pass(py, P1, False)
        hash_pass(pz, P2, False)

        def mask_body(i, carry):
            s = pl.ds(i * jnp.int32(_L), _L)
            hidx[s] = hidx[s] & jnp.int32(TABLE - 1)
            return carry

        lax.fori_loop(jnp.int32(0), jnp.int32(_PER_W // _L), mask_body,
                      jnp.int32(0))

        def fire_idx(j, carry):
            s = pl.ds(j * jnp.int32(_CHUNK), _CHUNK)
            pltpu.async_copy(buf.at[hidx.at[s]], vidx.at[s], sem_i)
            return carry

        lax.fori_loop(jnp.int32(0), jnp.int32(_NCHUNK), fire_idx,
                      jnp.int32(0))
        pltpu.make_async_copy(buf.at[pl.ds(jnp.int32(0), _PER_W)], vidx,
                              sem_i).wait()

        pltpu.sync_copy(tq.at[span], tsv)

        def enc_body(i, carry):
            s = pl.ds(i * jnp.int32(_L), _L)
            v = vidx[s]
            sidx[s] = jnp.maximum(v, jnp.int32(0))
            tm = jnp.remainder(tsv[s], jnp.int32(MODT))
            hidx[s] = jnp.where(v >= 0, tm, jnp.int32(MODT))
            return carry

        lax.fori_loop(jnp.int32(0), jnp.int32(_PER_W // _L), enc_body,
                      jnp.int32(0))
        pltpu.sync_copy(hidx, out_t.at[span])

        srows = (srows0, srows1)
        drows = (drows0, drows1)
        sems = ((sem_s0, sem_d0), (sem_s1, sem_d1))

        def fire(c, slot):
            s = pl.ds(c * jnp.int32(_CHUNK), _CHUNK)
            pltpu.async_copy(statf.at[sidx.at[s]], srows[slot], sems[slot][0])
            pltpu.async_copy(dynf.at[sidx.at[s]], drows[slot], sems[slot][1])

        def drain(slot):
            pltpu.make_async_copy(statf.at[pl.ds(jnp.int32(0), _CHUNK)],
                                  srows[slot], sems[slot][0]).wait()
            pltpu.make_async_copy(dynf.at[pl.ds(jnp.int32(0), _CHUNK)],
                                  drows[slot], sems[slot][1]).wait()

        def ship(c, slot):
            off = base + c * jnp.int32(_CHUNK)
            pltpu.sync_copy(srows[slot], out_s.at[pl.ds(off, _CHUNK)])
            pltpu.sync_copy(drows[slot], out_d.at[pl.ds(off, _CHUNK)])

        fire(jnp.int32(0), 0)
        fire(jnp.int32(1), 1)

        def pipe_body(k, carry):
            c0 = k * jnp.int32(2)
            drain(0)
            ship(c0, 0)

            @pl.when(k < _NCHUNK // 2 - 1)
            def _():
                fire(c0 + jnp.int32(2), 0)

            drain(1)
            ship(c0 + jnp.int32(1), 1)

            @pl.when(k < _NCHUNK // 2 - 1)
            def _():
                fire(c0 + jnp.int32(3), 1)

            return carry

        lax.fori_loop(jnp.int32(0), jnp.int32(_NCHUNK // 2), pipe_body,
                      jnp.int32(0))

    return sc_gather


_sc_gather = _sc_gather_build()

_BM = 1024
_INV_SQRT_HD = np.float32(1.0) / np.sqrt(np.float32(HD))
_TE_ROWS = MODT + 1


def _tc_fusion_body(t_ref, stat_ref, dyn_ref, te_ref,
                    wqkv1, wkv1, wo1, wqkv2, wkv2, wo2, out_ref):
    f32 = jnp.float32
    bf16 = jnp.bfloat16
    sd = lax.broadcasted_iota(jnp.int32, (DP, DP), 0) // HD
    sh = lax.broadcasted_iota(jnp.int32, (DP, DP), 1) // HD
    SSt = (sd == sh).astype(bf16)

    def fusion(a, b, Wqkv, Wkv, Wo):
        ab = a.astype(bf16)
        bb = b.astype(bf16)
        qkv = jnp.dot(ab, Wqkv[...], preferred_element_type=f32)
        kvb = jnp.dot(bb, Wkv[...], preferred_element_type=f32)
        q = qkv[:, :DP].astype(bf16)
        ka = qkv[:, DP:2 * DP].astype(bf16)
        va = qkv[:, 2 * DP:].astype(bf16)
        kb = kvb[:, :DP].astype(bf16)
        vb = kvb[:, DP:].astype(bf16)
        l0e = jnp.dot(q * ka, SSt, preferred_element_type=f32)
        l1e = jnp.dot(q * kb, SSt, preferred_element_type=f32)
        w1e = 1.0 / (1.0 + jnp.exp((l0e - l1e) * _INV_SQRT_HD))
        out = (w1e * (vb - va) + va).astype(bf16)
        return a + jnp.dot(out, Wo[...], preferred_element_type=f32)

    tm = t_ref[...].reshape(1, _BM)
    ohT = (jnp.broadcast_to(tm, (_TE_ROWS, _BM))
           == lax.broadcasted_iota(jnp.int32, (_TE_ROWS, _BM), 0)
           ).astype(bf16)
    raw_te = lax.dot_general(ohT, te_ref[...],
                             (((0,), (0,)), ((), ())),
                             preferred_element_type=f32)
    vld = raw_te[:, 120:121]
    lane = lax.broadcasted_iota(jnp.int32, (_BM, DP), 1)
    te = jnp.where(lane == 120, jnp.float32(0.0), raw_te)

    stat = stat_ref[...]
    dyn = dyn_ref[...]
    cond = fusion(dyn, te, wqkv1, wkv1, wo1)
    fused = fusion(stat, cond, wqkv2, wkv2, wo2)
    out_ref[...] = (fused * vld)[:, :D]


def _tc_fusion(t3, stat_g, dyn_g, te_ext, weights):
    grid = (_HALF // _BM,)
    z = np.int32(0)
    row_spec = pl.BlockSpec((_BM, DP), lambda i: (i, z))
    t_spec = pl.BlockSpec((1, 1, _BM), lambda i: (i, z, z))
    te_spec = pl.BlockSpec((_TE_ROWS, DP), lambda i: (z, z))
    w_specs = [pl.BlockSpec(w.shape, lambda i: (z, z)) for w in weights]
    out_spec = pl.BlockSpec((_BM, D), lambda i: (i, z))
    return pl.pallas_call(
        _tc_fusion_body,
        grid=grid,
        in_specs=[t_spec, row_spec, row_spec, te_spec] + w_specs,
        out_specs=out_spec,
        out_shape=jax.ShapeDtypeStruct((_HALF, D), jnp.float32),
    )(t3, stat_g, dyn_g, te_ext, *weights)


def kernel(query_pts, query_times, buffer_voxel_index, static_features,
           dynamic_features, time_embeddings,
           Wq1, Wk1, Wv1, Wo1, Wq2, Wk2, Wv2, Wo2):
    px = query_pts[:, 0]
    py = query_pts[:, 1]
    pz = query_pts[:, 2]
    t32 = query_times.astype(jnp.int32)
    buf32 = buffer_voxel_index.astype(jnp.int32)
    pad_w = ((0, 0), (0, DP - D))
    stat_p = jnp.pad(static_features, pad_w)
    dyn_p = jnp.pad(dynamic_features, pad_w)
    te_ext = jnp.concatenate([
        jnp.pad(time_embeddings, ((0, 0), (0, 1)), constant_values=1.0),
        jnp.zeros((MODT, DP - D - 1), jnp.float32),
    ], axis=1)
    te_ext = jnp.concatenate(
        [te_ext, jnp.zeros((1, DP), jnp.float32)], axis=0).astype(jnp.bfloat16)
    wp = {k: jnp.pad(w, (pad_w[1], pad_w[1])).astype(jnp.bfloat16)
          for k, w in dict(q1=Wq1, k1=Wk1, v1=Wv1, o1=Wo1,
                           q2=Wq2, k2=Wk2, v2=Wv2, o2=Wo2).items()}
    weights = [
        jnp.concatenate([wp["q1"], wp["k1"], wp["v1"]], axis=1),
        jnp.concatenate([wp["k1"], wp["v1"]], axis=1),
        wp["o1"],
        jnp.concatenate([wp["q2"], wp["k2"], wp["v2"]], axis=1),
        jnp.concatenate([wp["k2"], wp["v2"]], axis=1),
        wp["o2"],
    ]
    halves = []
    for h in range(2):
        sl = slice(h * _HALF, (h + 1) * _HALF)
        sg, dg, tenc = _sc_gather(px[sl], py[sl], pz[sl], t32[sl],
                                  buf32, stat_p, dyn_p)
        t3 = tenc.reshape(_HALF // _BM, 1, _BM)
        halves.append(_tc_fusion(t3, sg, dg, te_ext, weights))
    return jnp.concatenate(halves, axis=0)


# speedup vs baseline: 3.9292x; 1.0091x over previous
import functools

import jax
import jax.numpy as jnp
import numpy as np
from jax import lax
from jax.experimental import pallas as pl
from jax.experimental.pallas import tpu as pltpu
from jax.experimental.pallas import tpu_sc as plsc

RES = 0.1
TABLE = 2 ** 20
D = 120
DP = 128
H = 8
HD = D // H
MODT = 201
M = 262144
P0 = 73856093
P1 = 19349669
P2 = 83492791

_NC = 2
_NS = 16
_L = 16
_NW = _NC * _NS
_HALF = M // 4
_PER_W = _HALF // _NW
_CHUNK = 128
_NCHUNK = _PER_W // _CHUNK


def _sc_gather_build():
    mesh = plsc.VectorSubcoreMesh(core_axis_name="c", subcore_axis_name="s")

    @functools.partial(
        pl.kernel,
        mesh=mesh,
        out_type=[
            jax.ShapeDtypeStruct((_HALF, DP), jnp.float32),
            jax.ShapeDtypeStruct((_HALF, DP), jnp.float32),
            jax.ShapeDtypeStruct((_HALF,), jnp.int32),
        ],
        scratch_types=[
            pltpu.VMEM((_PER_W,), jnp.float32),
            pltpu.VMEM((_PER_W,), jnp.int32),
            pltpu.VMEM((_PER_W,), jnp.int32),
            pltpu.VMEM((_PER_W,), jnp.int32),
            pltpu.VMEM((_PER_W,), jnp.int32),
            pltpu.VMEM((_CHUNK, DP), jnp.float32),
            pltpu.VMEM((_CHUNK, DP), jnp.float32),
            pltpu.VMEM((_CHUNK, DP), jnp.float32),
            pltpu.VMEM((_CHUNK, DP), jnp.float32),
            pltpu.SemaphoreType.DMA,
            pltpu.SemaphoreType.DMA,
            pltpu.SemaphoreType.DMA,
            pltpu.SemaphoreType.DMA,
            pltpu.SemaphoreType.DMA,
        ],
    )
    def sc_gather(px, py, pz, tq, buf, statf, dynf,
                  out_s, out_d, out_t,
                  coords, hidx, vidx, sidx, tsv,
                  srows0, srows1, drows0, drows1,
                  sem_i, sem_s0, sem_d0, sem_s1, sem_d1):
        wid = (lax.axis_index("s") * jnp.int32(_NC)
               + lax.axis_index("c")).astype(jnp.int32)
        base = wid * jnp.int32(_PER_W)
        span = pl.ds(base, _PER_W)

        def hash_pass(src, prime, first):
            pltpu.sync_copy(src.at[span], coords)

            def body(i, carry):
                s = pl.ds(i * jnp.int32(_L), _L)
                g = (coords[s] / RES).astype(jnp.int32) * jnp.int32(prime)
                hidx[s] = g if first else hidx[s] + g
                return carry

            lax.fori_loop(jnp.int32(0), jnp.int32(_PER_W // _L), body,
                          jnp.int32(0))

        hash_pass(px, P0, True)
        hash_pass(py, P1, False)
        hash_pass(pz, P2, False)

        def mask_body(i, carry):
            s = pl.ds(i * jnp.int32(_L), _L)
            hidx[s] = hidx[s] & jnp.int32(TABLE - 1)
            return carry

        lax.fori_loop(jnp.int32(0), jnp.int32(_PER_W // _L), mask_body,
                      jnp.int32(0))

        def fire_idx(j, carry):
            s = pl.ds(j * jnp.int32(_CHUNK), _CHUNK)
            pltpu.async_copy(buf.at[hidx.at[s]], vidx.at[s], sem_i)
            return carry

        lax.fori_loop(jnp.int32(0), jnp.int32(_NCHUNK), fire_idx,
                      jnp.int32(0))
        pltpu.make_async_copy(buf.at[pl.ds(jnp.int32(0), _PER_W)], vidx,
                              sem_i).wait()

        pltpu.sync_copy(tq.at[span], tsv)

        def enc_body(i, carry):
            s = pl.ds(i * jnp.int32(_L), _L)
            v = vidx[s]
            sidx[s] = jnp.maximum(v, jnp.int32(0))
            tm = jnp.remainder(tsv[s], jnp.int32(MODT))
            hidx[s] = jnp.where(v >= 0, tm, jnp.int32(MODT))
            return carry

        lax.fori_loop(jnp.int32(0), jnp.int32(_PER_W // _L), enc_body,
                      jnp.int32(0))
        pltpu.sync_copy(hidx, out_t.at[span])

        srows = (srows0, srows1)
        drows = (drows0, drows1)
        sems = ((sem_s0, sem_d0), (sem_s1, sem_d1))

        def fire(c, slot):
            s = pl.ds(c * jnp.int32(_CHUNK), _CHUNK)
            pltpu.async_copy(statf.at[sidx.at[s]], srows[slot], sems[slot][0])
            pltpu.async_copy(dynf.at[sidx.at[s]], drows[slot], sems[slot][1])

        def drain(slot):
            pltpu.make_async_copy(statf.at[pl.ds(jnp.int32(0), _CHUNK)],
                                  srows[slot], sems[slot][0]).wait()
            pltpu.make_async_copy(dynf.at[pl.ds(jnp.int32(0), _CHUNK)],
                                  drows[slot], sems[slot][1]).wait()

        def ship(c, slot):
            off = base + c * jnp.int32(_CHUNK)
            pltpu.sync_copy(srows[slot], out_s.at[pl.ds(off, _CHUNK)])
            pltpu.sync_copy(drows[slot], out_d.at[pl.ds(off, _CHUNK)])

        fire(jnp.int32(0), 0)
        fire(jnp.int32(1), 1)

        def pipe_body(k, carry):
            c0 = k * jnp.int32(2)
            drain(0)
            ship(c0, 0)

            @pl.when(k < _NCHUNK // 2 - 1)
            def _():
                fire(c0 + jnp.int32(2), 0)

            drain(1)
            ship(c0 + jnp.int32(1), 1)

            @pl.when(k < _NCHUNK // 2 - 1)
            def _():
                fire(c0 + jnp.int32(3), 1)

            return carry

        lax.fori_loop(jnp.int32(0), jnp.int32(_NCHUNK // 2), pipe_body,
                      jnp.int32(0))

    return sc_gather


_sc_gather = _sc_gather_build()

_BM = 1024
_INV_SQRT_HD = np.float32(1.0) / np.sqrt(np.float32(HD))
_TE_ROWS = MODT + 1


def _tc_fusion_body(t_ref, stat_ref, dyn_ref, te_ref,
                    wqkv1, wkv1, wo1, wqkv2, wkv2, wo2, out_ref):
    f32 = jnp.float32
    bf16 = jnp.bfloat16
    sd = lax.broadcasted_iota(jnp.int32, (DP, DP), 0) // HD
    sh = lax.broadcasted_iota(jnp.int32, (DP, DP), 1) // HD
    SSt = (sd == sh).astype(bf16)

    def fusion(a, b, Wqkv, Wkv, Wo):
        ab = a.astype(bf16)
        bb = b.astype(bf16)
        qkv = jnp.dot(ab, Wqkv[...], preferred_element_type=f32)
        kvb = jnp.dot(bb, Wkv[...], preferred_element_type=f32)
        q = qkv[:, :DP].astype(bf16)
        ka = qkv[:, DP:2 * DP].astype(bf16)
        va = qkv[:, 2 * DP:].astype(bf16)
        kb = kvb[:, :DP].astype(bf16)
        vb = kvb[:, DP:].astype(bf16)
        l0e = jnp.dot(q * ka, SSt, preferred_element_type=f32)
        l1e = jnp.dot(q * kb, SSt, preferred_element_type=f32)
        w1e = 1.0 / (1.0 + jnp.exp((l0e - l1e) * _INV_SQRT_HD))
        out = (w1e * (vb - va) + va).astype(bf16)
        return a + jnp.dot(out, Wo[...], preferred_element_type=f32)

    tm = t_ref[...].reshape(1, _BM)
    ohT = (jnp.broadcast_to(tm, (_TE_ROWS, _BM))
           == lax.broadcasted_iota(jnp.int32, (_TE_ROWS, _BM), 0)
           ).astype(bf16)
    raw_te = lax.dot_general(ohT, te_ref[...],
                             (((0,), (0,)), ((), ())),
                             preferred_element_type=f32)
    vld = raw_te[:, 120:121]
    lane = lax.broadcasted_iota(jnp.int32, (_BM, DP), 1)
    te = jnp.where(lane == 120, jnp.float32(0.0), raw_te)

    stat = stat_ref[...]
    dyn = dyn_ref[...]
    cond = fusion(dyn, te, wqkv1, wkv1, wo1)
    fused = fusion(stat, cond, wqkv2, wkv2, wo2)
    out_ref[...] = (fused * vld)[:, :D]


def _tc_fusion(t3, stat_g, dyn_g, te_ext, weights):
    grid = (_HALF // _BM,)
    z = np.int32(0)
    row_spec = pl.BlockSpec((_BM, DP), lambda i: (i, z))
    t_spec = pl.BlockSpec((1, 1, _BM), lambda i: (i, z, z))
    te_spec = pl.BlockSpec((_TE_ROWS, DP), lambda i: (z, z))
    w_specs = [pl.BlockSpec(w.shape, lambda i: (z, z)) for w in weights]
    out_spec = pl.BlockSpec((_BM, D), lambda i: (i, z))
    return pl.pallas_call(
        _tc_fusion_body,
        grid=grid,
        in_specs=[t_spec, row_spec, row_spec, te_spec] + w_specs,
        out_specs=out_spec,
        out_shape=jax.ShapeDtypeStruct((_HALF, D), jnp.float32),
    )(t3, stat_g, dyn_g, te_ext, *weights)


def kernel(query_pts, query_times, buffer_voxel_index, static_features,
           dynamic_features, time_embeddings,
           Wq1, Wk1, Wv1, Wo1, Wq2, Wk2, Wv2, Wo2):
    px = query_pts[:, 0]
    py = query_pts[:, 1]
    pz = query_pts[:, 2]
    t32 = query_times.astype(jnp.int32)
    buf32 = buffer_voxel_index.astype(jnp.int32)
    pad_w = ((0, 0), (0, DP - D))
    stat_p = jnp.pad(static_features, pad_w)
    dyn_p = jnp.pad(dynamic_features, pad_w)
    te_ext = jnp.concatenate([
        jnp.pad(time_embeddings, ((0, 0), (0, 1)), constant_values=1.0),
        jnp.zeros((MODT, DP - D - 1), jnp.float32),
    ], axis=1)
    te_ext = jnp.concatenate(
        [te_ext, jnp.zeros((1, DP), jnp.float32)], axis=0).astype(jnp.bfloat16)
    wp = {k: jnp.pad(w, (pad_w[1], pad_w[1])).astype(jnp.bfloat16)
          for k, w in dict(q1=Wq1, k1=Wk1, v1=Wv1, o1=Wo1,
                           q2=Wq2, k2=Wk2, v2=Wv2, o2=Wo2).items()}
    weights = [
        jnp.concatenate([wp["q1"], wp["k1"], wp["v1"]], axis=1),
        jnp.concatenate([wp["k1"], wp["v1"]], axis=1),
        wp["o1"],
        jnp.concatenate([wp["q2"], wp["k2"], wp["v2"]], axis=1),
        jnp.concatenate([wp["k2"], wp["v2"]], axis=1),
        wp["o2"],
    ]
    halves = []
    for h in range(4):
        sl = slice(h * _HALF, (h + 1) * _HALF)
        sg, dg, tenc = _sc_gather(px[sl], py[sl], pz[sl], t32[sl],
                                  buf32, stat_p, dyn_p)
        t3 = tenc.reshape(_HALF // _BM, 1, _BM)
        halves.append(_tc_fusion(t3, sg, dg, te_ext, weights))
    return jnp.concatenate(halves, axis=0)
